# baseline (speedup 1.0000x reference)
import functools

import jax
import jax.numpy as jnp
from jax import lax
from jax.experimental import pallas as pl
from jax.experimental.pallas import tpu as pltpu

_ALPHA = 0.01


def _body(nblk, scale, idx_ref, probs_ref, out_ref, acc8, acc_p):
    i = pl.program_id(0)

    @pl.when(i == 0)
    def _():
        acc8[...] = jnp.zeros_like(acc8)
        acc_p[...] = jnp.zeros_like(acc_p)

    idxb = idx_ref[...]
    iota8 = lax.broadcasted_iota(jnp.int32, (8, 1), 0)
    c8 = jnp.zeros((8, 8), jnp.float32)
    for r in range(8):
        strip = idxb[r:r + 1, :]
        hm = ((strip >> 3) == iota8).astype(jnp.float32)
        lm = ((strip & 7) == iota8).astype(jnp.float32)
        c8 += lax.dot_general(
            hm, lm, (((1,), (1,)), ((), ())),
            preferred_element_type=jnp.float32)
    acc8[...] += c8

    pb = probs_ref[...]
    ones = jnp.ones((1, pb.shape[0]), jnp.float32)
    acc_p[...] += lax.dot_general(
        ones, pb, (((1,), (0,)), ((), ())),
        preferred_element_type=jnp.float32)

    @pl.when(i == nblk - 1)
    def _():
        ps = acc_p[0:1, 0:64] + acc_p[0:1, 64:128]
        e_row = lax.broadcasted_iota(jnp.int32, (8, 64), 1)
        h_row = lax.broadcasted_iota(jnp.int32, (8, 64), 0)
        hmask = ((e_row >> 3) == h_row).astype(jnp.float32)
        e_col = lax.broadcasted_iota(jnp.int32, (64, 8), 0)
        l_col = lax.broadcasted_iota(jnp.int32, (64, 8), 1)
        lmask = ((e_col & 7) == l_col).astype(jnp.float32)
        psum8 = lax.dot_general(
            hmask * ps, lmask, (((1,), (0,)), ((), ())),
            preferred_element_type=jnp.float32)
        out_ref[...] = jnp.sum(acc8[...] * psum8, keepdims=True) * scale


def kernel(indices, weights, probs, n_experts):
    del weights, n_experts
    T, K = indices.shape
    E = probs.shape[-1]
    nblk = 8
    n_flat = T * K
    lb = n_flat // (nblk * 8)
    idx2d = indices.astype(jnp.int32).reshape(nblk * 8, lb)
    probs2 = probs.reshape(T * E // 128, 128)
    rows_p = probs2.shape[0] // nblk
    scale = _ALPHA * E / (float(T) * float(T))

    out = pl.pallas_call(
        functools.partial(_body, nblk, scale),
        grid=(nblk,),
        in_specs=[
            pl.BlockSpec((8, lb), lambda i: (i, 0)),
            pl.BlockSpec((rows_p, 128), lambda i: (i, 0)),
        ],
        out_specs=pl.BlockSpec((1, 1), lambda i: (0, 0)),
        out_shape=jax.ShapeDtypeStruct((1, 1), jnp.float32),
        scratch_shapes=[
            pltpu.VMEM((8, 8), jnp.float32),
            pltpu.VMEM((1, 128), jnp.float32),
        ],
        compiler_params=pltpu.CompilerParams(
            dimension_semantics=("arbitrary",),
        ),
    )(idx2d, probs2)
    return out[0, 0]


# speedup vs baseline: 1.3140x; 1.3140x over previous
import functools

import jax
import jax.numpy as jnp
from jax import lax
from jax.experimental import pallas as pl
from jax.experimental.pallas import tpu as pltpu

_ALPHA = 0.01


def _body(nblk, scale, idx_ref, p0, p1, p2, p3, out_ref, acc8, acc_p):
    i = pl.program_id(0)

    @pl.when(i == 0)
    def _():
        acc8[...] = jnp.zeros_like(acc8)
        acc_p[...] = jnp.zeros_like(acc_p)

    idxb = idx_ref[...]
    iota8 = lax.broadcasted_iota(jnp.int32, (8, 1), 0)
    c8 = jnp.zeros((8, 8), jnp.float32)
    for r in range(8):
        strip = idxb[r:r + 1, :]
        hm = ((strip >> 3) == iota8).astype(jnp.float32)
        lm = ((strip & 7) == iota8).astype(jnp.float32)
        c8 += lax.dot_general(
            hm, lm, (((1,), (1,)), ((), ())),
            preferred_element_type=jnp.float32)
    acc8[...] += c8

    rows = p0.shape[0]
    ones = jnp.ones((1, rows), jnp.float32)
    psum = jnp.zeros((1, 64), jnp.float32)
    for p in (p0, p1, p2, p3):
        psum += lax.dot_general(
            ones, p[...], (((1,), (0,)), ((), ())),
            preferred_element_type=jnp.float32)
    acc_p[...] += psum

    @pl.when(i == nblk - 1)
    def _():
        ps = acc_p[...]
        e_row = lax.broadcasted_iota(jnp.int32, (8, 64), 1)
        h_row = lax.broadcasted_iota(jnp.int32, (8, 64), 0)
        hmask = ((e_row >> 3) == h_row).astype(jnp.float32)
        e_col = lax.broadcasted_iota(jnp.int32, (64, 8), 0)
        l_col = lax.broadcasted_iota(jnp.int32, (64, 8), 1)
        lmask = ((e_col & 7) == l_col).astype(jnp.float32)
        psum8 = lax.dot_general(
            hmask * ps, lmask, (((1,), (0,)), ((), ())),
            preferred_element_type=jnp.float32)
        out_ref[...] = jnp.sum(acc8[...] * psum8, keepdims=True) * scale


def kernel(indices, weights, probs, n_experts):
    del weights, n_experts
    T, K = indices.shape
    E = probs.shape[-1]
    nblk = 2
    nstream = 4
    n_flat = T * K
    lb = n_flat // (nblk * 8)
    idx2d = indices.astype(jnp.int32).reshape(nblk * 8, lb)
    rows_p = T // (nblk * nstream)
    scale = _ALPHA * E / (float(T) * float(T))

    def mkp(s):
        return pl.BlockSpec((rows_p, E), lambda i, s=s: (s * nblk + i, 0))

    out = pl.pallas_call(
        functools.partial(_body, nblk, scale),
        grid=(nblk,),
        in_specs=[pl.BlockSpec((8, lb), lambda i: (i, 0))]
        + [mkp(s) for s in range(nstream)],
        out_specs=pl.BlockSpec((1, 1), lambda i: (0, 0)),
        out_shape=jax.ShapeDtypeStruct((1, 1), jnp.float32),
        scratch_shapes=[
            pltpu.VMEM((8, 8), jnp.float32),
            pltpu.VMEM((1, E), jnp.float32),
        ],
        compiler_params=pltpu.CompilerParams(
            dimension_semantics=("arbitrary",),
        ),
    )(idx2d, probs, probs, probs, probs)
    return out[0, 0]


# speedup vs baseline: 8.6674x; 6.5962x over previous
import functools

import jax
import jax.numpy as jnp
from jax import lax
from jax.experimental import pallas as pl
from jax.experimental.pallas import tpu as pltpu

_ALPHA = 0.01


def _body(nblk, scale, idx_ref, pt_ref, out_ref, acc8, acc_p):
    i = pl.program_id(0)

    @pl.when(i == 0)
    def _():
        acc8[...] = jnp.zeros_like(acc8)
        acc_p[...] = jnp.zeros_like(acc_p)

    idxb = idx_ref[...]
    iota8 = lax.broadcasted_iota(jnp.int32, (8, 1), 0)
    c8 = jnp.zeros((8, 8), jnp.float32)
    for r in range(8):
        strip = idxb[r:r + 1, :]
        hm = ((strip >> 3) == iota8).astype(jnp.float32)
        lm = ((strip & 7) == iota8).astype(jnp.float32)
        c8 += lax.dot_general(
            hm, lm, (((1,), (1,)), ((), ())),
            preferred_element_type=jnp.float32)
    acc8[...] += c8

    acc_p[...] += jnp.sum(pt_ref[...], axis=1, keepdims=True)

    @pl.when(i == nblk - 1)
    def _():
        e_col = lax.broadcasted_iota(jnp.int32, (64, 8), 0)
        x_col = lax.broadcasted_iota(jnp.int32, (64, 8), 1)
        m_h = ((e_col >> 3) == x_col).astype(jnp.float32)
        m_l = ((e_col & 7) == x_col).astype(jnp.float32)
        tmp = lax.dot_general(
            m_h, acc8[...], (((1,), (0,)), ((), ())),
            preferred_element_type=jnp.float32)
        cnt64 = jnp.sum(tmp * m_l, axis=1, keepdims=True)
        out_ref[...] = jnp.sum(cnt64 * acc_p[...], keepdims=True) * scale


def kernel(indices, weights, probs, n_experts):
    del weights, n_experts
    T, K = indices.shape
    E = probs.shape[-1]
    nblk = 2
    lb = T // nblk
    idxT = indices.astype(jnp.int32).T
    probsT = probs.T
    scale = _ALPHA * E / (float(T) * float(T))

    out = pl.pallas_call(
        functools.partial(_body, nblk, scale),
        grid=(nblk,),
        in_specs=[
            pl.BlockSpec((K, lb), lambda i: (0, i)),
            pl.BlockSpec((E, lb), lambda i: (0, i)),
        ],
        out_specs=pl.BlockSpec((1, 1), lambda i: (0, 0)),
        out_shape=jax.ShapeDtypeStruct((1, 1), jnp.float32),
        scratch_shapes=[
            pltpu.VMEM((8, 8), jnp.float32),
            pltpu.VMEM((E, 1), jnp.float32),
        ],
        compiler_params=pltpu.CompilerParams(
            dimension_semantics=("arbitrary",),
        ),
    )(idxT, probsT)
    return out[0, 0]
